# baseline (speedup 1.0000x reference)
import functools
import jax
import jax.numpy as jnp
from jax import lax
from jax.experimental import pallas as pl
from jax.experimental.pallas import tpu as pltpu
from jax.experimental.pallas import tpu_sc as plsc

_B, _S, _DIM = 2, 2048, 1024
_E, _FFN, _TOPK = 16, 2048, 128
_EPS = 1e-5


def _router_body(x_ref, rw_ref, logits_ref, probt_ref):
    x = x_ref[0]
    rw = rw_ref[...]
    logits = lax.dot_general(x, rw, (((1,), (1,)), ((), ())),
                             preferred_element_type=jnp.float32)
    logits_ref[0] = logits
    m = jnp.max(logits, axis=-1, keepdims=True)
    ex = jnp.exp(logits - m)
    probs = ex / jnp.sum(ex, axis=-1, keepdims=True)
    probt_ref[0] = probs.T


def _router(inputs, router_w):
    return pl.pallas_call(
        _router_body,
        grid=(_B,),
        in_specs=[
            pl.BlockSpec((1, _S, _DIM), lambda b: (b, 0, 0)),
            pl.BlockSpec((_E, _DIM), lambda b: (0, 0)),
        ],
        out_specs=[
            pl.BlockSpec((1, _S, _E), lambda b: (b, 0, 0)),
            pl.BlockSpec((1, _E, _S), lambda b: (b, 0, 0)),
        ],
        out_shape=[
            jax.ShapeDtypeStruct((_B, _S, _E), jnp.float32),
            jax.ShapeDtypeStruct((_B, _E, _S), jnp.float32),
        ],
    )(inputs, router_w)


def _ffn_body(xg_ref, w_ref, fc1w_ref, fc1b_ref, lng_ref, lnb_ref,
              fc2w_ref, fc2b_ref, out_ref):
    xg = xg_ref[0].astype(jnp.bfloat16)
    h = lax.dot_general(xg, fc1w_ref[0].astype(jnp.bfloat16),
                        (((1,), (1,)), ((), ())),
                        preferred_element_type=jnp.float32)
    h = h + fc1b_ref[0]
    h = 0.5 * h * (1.0 + lax.erf(h * 0.7071067811865476))
    mu = jnp.mean(h, axis=-1, keepdims=True)
    var = jnp.mean(jnp.square(h - mu), axis=-1, keepdims=True)
    h = (h - mu) * lax.rsqrt(var + _EPS) * lng_ref[0] + lnb_ref[0]
    y = lax.dot_general(fc2w_ref[0].astype(jnp.bfloat16),
                        h.astype(jnp.bfloat16),
                        (((1,), (1,)), ((), ())),
                        preferred_element_type=jnp.float32)
    y = y + fc2b_ref[0].reshape(_DIM, 1)
    out_ref[0] = y * w_ref[0]


def _ffn(xg, wsel, fc1_w, fc1_b, ln_g, ln_b, fc2_w, fc2_b):
    grid = (_B * _E,)
    return pl.pallas_call(
        _ffn_body,
        grid=grid,
        in_specs=[
            pl.BlockSpec((1, _TOPK, _DIM), lambda i: (i, 0, 0)),
            pl.BlockSpec((1, 1, _TOPK), lambda i: (i, 0, 0)),
            pl.BlockSpec((1, _FFN, _DIM), lambda i: (i % _E, 0, 0)),
            pl.BlockSpec((1, 1, _FFN), lambda i: (i % _E, 0, 0)),
            pl.BlockSpec((1, 1, _FFN), lambda i: (i % _E, 0, 0)),
            pl.BlockSpec((1, 1, _FFN), lambda i: (i % _E, 0, 0)),
            pl.BlockSpec((1, _DIM, _FFN), lambda i: (i % _E, 0, 0)),
            pl.BlockSpec((1, 1, _DIM), lambda i: (i % _E, 0, 0)),
        ],
        out_specs=pl.BlockSpec((1, _DIM, _TOPK), lambda i: (i // _E, 0, i % _E)),
        out_shape=jax.ShapeDtypeStruct((_B, _DIM, _E * _TOPK), jnp.float32),
    )(xg, wsel, fc1_w, fc1_b, ln_g, ln_b, fc2_w, fc2_b)


_NC, _NS = 2, 16
_CH = _S // _NS
_NW = _NC * _NS
_TC = _DIM // _NW
_RC = 512


def _topk_gather_body(probt_ref, x_ref, sel_ref, w_ref, xg_ref,
                      pv, selb, wvb, rows_g, sem):
    c = lax.axis_index("c")
    s = lax.axis_index("s")
    pltpu.sync_copy(probt_ref.at[c, s], pv)

    def _count(thr):
        def cbody(j, acc):
            bits = plsc.bitcast(pv[0, pl.ds(j * 16, 16)], jnp.int32)
            return acc + jnp.where(bits >= thr, 1, 0)
        acc = lax.fori_loop(0, _S // 16, cbody,
                            jnp.zeros((16,), jnp.int32))
        return jnp.sum(acc)

    def _bis(i, lohi):
        lo, hi = lohi
        mid = (lo + hi) // 2
        big = _count(mid) >= _TOPK
        return (jnp.where(big, mid, lo), jnp.where(big, hi, mid))
    tstar, _ = lax.fori_loop(0, 31, _bis,
                             (jnp.int32(0), jnp.int32(0x3F800001)))
    need = _TOPK - _count(tstar + 1)

    def _comp(j, st):
        pos, taken = st
        v = pv[0, pl.ds(j * 16, 16)]
        bits = plsc.bitcast(v, jnp.int32)
        m_gt = bits > tstar
        m_eq = bits == tstar
        eqc = plsc.cumsum(jnp.where(m_eq, 1, 0))
        take_eq = m_eq & ((eqc + taken) <= need)
        selm = m_gt | take_eq
        idxs = lax.iota(jnp.int32, 16) + j * 16
        plsc.store_compressed(selb.at[pl.ds(pos, 16)], idxs, mask=selm)
        plsc.store_compressed(wvb.at[pl.ds(pos, 16)], v, mask=selm)
        return (pos + jnp.sum(jnp.where(selm, 1, 0)),
                taken + jnp.sum(jnp.where(take_eq, 1, 0)))
    lax.fori_loop(0, _S // 16, _comp, (jnp.int32(0), jnp.int32(0)))

    pltpu.sync_copy(selb.at[pl.ds(0, _TOPK)], sel_ref.at[c, s, 0])
    pltpu.sync_copy(wvb.at[pl.ds(0, _TOPK)], w_ref.at[c, s, 0])
    for h in range(2):
        pltpu.async_copy(x_ref.at[c].at[selb.at[pl.ds(h * 64, 64)]],
                         rows_g, sem).wait()
        pltpu.sync_copy(rows_g, xg_ref.at[c, s, pl.ds(h * 64, 64)])


def _topk_gather(probt, x):
    mesh = plsc.VectorSubcoreMesh(core_axis_name="c", subcore_axis_name="s",
                                  num_cores=_NC, num_subcores=_NS)
    f = pl.kernel(
        _topk_gather_body,
        out_type=[
            jax.ShapeDtypeStruct((_B, _E, 1, _TOPK), jnp.int32),
            jax.ShapeDtypeStruct((_B, _E, 1, _TOPK), jnp.float32),
            jax.ShapeDtypeStruct((_B, _E, _TOPK, _DIM), jnp.float32),
        ],
        mesh=mesh,
        scratch_types=[
            pltpu.VMEM((1, _S), jnp.float32),
            pltpu.VMEM((160,), jnp.int32),
            pltpu.VMEM((160,), jnp.float32),
            pltpu.VMEM((64, _DIM), jnp.float32),
            pltpu.SemaphoreType.DMA,
        ],
        compiler_params=pltpu.CompilerParams(needs_layout_passes=False),
    )
    return f(probt.reshape(_B, _E, 1, _S), x)


def _scatter_body(contrib_ref, sel_ref, out_ref, out_v, cbuf, idx_v):
    c = lax.axis_index("c")
    s = lax.axis_index("s")
    wid = s * _NC + c
    col0 = wid * _TC
    for b in range(_B):
        @plsc.parallel_loop(0, _S, unroll=8)
        def _zero(i):
            out_v[i, pl.ds(0, 16)] = jnp.zeros((16,), jnp.float32)
            out_v[i, pl.ds(16, 16)] = jnp.zeros((16,), jnp.float32)

        pltpu.sync_copy(sel_ref.at[b], idx_v)

        def _sub(sub, _):
            pltpu.sync_copy(
                contrib_ref.at[b, pl.ds(col0, _TC), pl.ds(sub * _RC, _RC)],
                cbuf)
            for r4 in range(_RC // _TOPK):
                e = sub * (_RC // _TOPK) + r4
                cb = r4 * _TOPK
                for k in range(_TOPK // 16):
                    idxg = idx_v[e, pl.ds(k * 16, 16)]
                    for colc in range(_TC):
                        x = cbuf[colc, pl.ds(cb + k * 16, 16)]
                        plsc.addupdate_scatter(
                            out_v, [idxg, jnp.full((16,), colc, jnp.int32)],
                            x)
            return 0
        lax.fori_loop(0, _E * _TOPK // _RC, _sub, 0)
        pltpu.sync_copy(out_v, out_ref.at[b, :, pl.ds(col0, _TC)])


def _scatter_add(contrib_t, sel):
    mesh = plsc.VectorSubcoreMesh(core_axis_name="c", subcore_axis_name="s",
                                  num_cores=_NC, num_subcores=_NS)
    f = pl.kernel(
        _scatter_body,
        out_type=jax.ShapeDtypeStruct((_B, _S, _DIM), jnp.float32),
        mesh=mesh,
        scratch_types=[
            pltpu.VMEM((_S, _TC), jnp.float32),
            pltpu.VMEM((_TC, _RC), jnp.float32),
            pltpu.VMEM((_E, _TOPK), jnp.int32),
        ],
        compiler_params=pltpu.CompilerParams(use_tc_tiling_on_sc=False,
                                             needs_layout_passes=False),
    )
    return f(contrib_t, sel)


def kernel(inputs, router_w, fc1_w, fc1_b, ln_g, ln_b, fc2_w, fc2_b):
    router_logits, probt = _router(inputs, router_w)
    selected, weights, xg = _topk_gather(probt, inputs)
    selected = selected.reshape(_B, _E, _TOPK)

    contrib_t = _ffn(xg.reshape(_B * _E, _TOPK, _DIM),
                     weights.reshape(_B * _E, 1, _TOPK),
                     fc1_w, fc1_b.reshape(_E, 1, _FFN),
                     ln_g.reshape(_E, 1, _FFN), ln_b.reshape(_E, 1, _FFN),
                     fc2_w, fc2_b.reshape(_E, 1, _DIM))

    out = _scatter_add(contrib_t, selected)
    return out, router_logits


# speedup vs baseline: 1.1205x; 1.1205x over previous
import functools
import jax
import jax.numpy as jnp
from jax import lax
from jax.experimental import pallas as pl
from jax.experimental.pallas import tpu as pltpu
from jax.experimental.pallas import tpu_sc as plsc

_B, _S, _DIM = 2, 2048, 1024
_E, _FFN, _TOPK = 16, 2048, 128
_EPS = 1e-5


def _router_body(x_ref, rw_ref, logits_ref, probt_ref):
    x = x_ref[0]
    rw = rw_ref[...]
    logits = lax.dot_general(x, rw, (((1,), (1,)), ((), ())),
                             preferred_element_type=jnp.float32)
    logits_ref[0] = logits
    m = jnp.max(logits, axis=-1, keepdims=True)
    ex = jnp.exp(logits - m)
    probs = ex / jnp.sum(ex, axis=-1, keepdims=True)
    probt_ref[0] = probs.T


def _router(inputs, router_w):
    return pl.pallas_call(
        _router_body,
        grid=(_B,),
        in_specs=[
            pl.BlockSpec((1, _S, _DIM), lambda b: (b, 0, 0)),
            pl.BlockSpec((_E, _DIM), lambda b: (0, 0)),
        ],
        out_specs=[
            pl.BlockSpec((1, _S, _E), lambda b: (b, 0, 0)),
            pl.BlockSpec((1, _E, _S), lambda b: (b, 0, 0)),
        ],
        out_shape=[
            jax.ShapeDtypeStruct((_B, _S, _E), jnp.float32),
            jax.ShapeDtypeStruct((_B, _E, _S), jnp.float32),
        ],
    )(inputs, router_w)


def _ffn_body(xg_ref, w_ref, fc1w_ref, fc1b_ref, lng_ref, lnb_ref,
              fc2w_ref, fc2b_ref, out_ref):
    xg = xg_ref[0].astype(jnp.bfloat16)
    h = lax.dot_general(xg, fc1w_ref[0].astype(jnp.bfloat16),
                        (((1,), (1,)), ((), ())),
                        preferred_element_type=jnp.float32)
    h = h + fc1b_ref[0]
    h = 0.5 * h * (1.0 + lax.erf(h * 0.7071067811865476))
    mu = jnp.mean(h, axis=-1, keepdims=True)
    var = jnp.mean(jnp.square(h - mu), axis=-1, keepdims=True)
    h = (h - mu) * lax.rsqrt(var + _EPS) * lng_ref[0] + lnb_ref[0]
    y = lax.dot_general(fc2w_ref[0].astype(jnp.bfloat16),
                        h.astype(jnp.bfloat16),
                        (((1,), (1,)), ((), ())),
                        preferred_element_type=jnp.float32)
    y = y + fc2b_ref[0].reshape(_DIM, 1)
    out_ref[0] = y * w_ref[0]


def _ffn(xg, wsel, fc1_w, fc1_b, ln_g, ln_b, fc2_w, fc2_b):
    grid = (_B * _E,)
    return pl.pallas_call(
        _ffn_body,
        grid=grid,
        in_specs=[
            pl.BlockSpec((1, _TOPK, _DIM), lambda i: (i, 0, 0)),
            pl.BlockSpec((1, 1, _TOPK), lambda i: (i, 0, 0)),
            pl.BlockSpec((1, _FFN, _DIM), lambda i: (i % _E, 0, 0)),
            pl.BlockSpec((1, 1, _FFN), lambda i: (i % _E, 0, 0)),
            pl.BlockSpec((1, 1, _FFN), lambda i: (i % _E, 0, 0)),
            pl.BlockSpec((1, 1, _FFN), lambda i: (i % _E, 0, 0)),
            pl.BlockSpec((1, _DIM, _FFN), lambda i: (i % _E, 0, 0)),
            pl.BlockSpec((1, 1, _DIM), lambda i: (i % _E, 0, 0)),
        ],
        out_specs=pl.BlockSpec((1, _DIM, _TOPK), lambda i: (i // _E, 0, i % _E)),
        out_shape=jax.ShapeDtypeStruct((_B, _DIM, _E * _TOPK), jnp.float32),
    )(xg, wsel, fc1_w, fc1_b, ln_g, ln_b, fc2_w, fc2_b)


_NC, _NS = 2, 16
_CH = _S // _NS
_NW = _NC * _NS
_TC = _DIM // _NW
_RC = 512


def _topk_gather_body(probt_ref, x_ref, sel_ref, w_ref, xg_ref,
                      pv, selb, wvb, rows_g, sem):
    c = lax.axis_index("c")
    s = lax.axis_index("s")
    pltpu.sync_copy(probt_ref.at[c, s], pv)

    def _count(thr):
        def cbody(j, acc):
            bits = plsc.bitcast(pv[0, pl.ds(j * 16, 16)], jnp.int32)
            return acc + jnp.where(bits >= thr, 1, 0)
        acc = lax.fori_loop(0, _S // 16, cbody,
                            jnp.zeros((16,), jnp.int32))
        return jnp.sum(acc)

    def _bis(i, lohi):
        lo, hi = lohi
        mid = (lo + hi) // 2
        big = _count(mid) >= _TOPK
        return (jnp.where(big, mid, lo), jnp.where(big, hi, mid))
    tstar, _ = lax.fori_loop(0, 31, _bis,
                             (jnp.int32(0), jnp.int32(0x3F800001)))
    need = _TOPK - _count(tstar + 1)

    def _comp(j, st):
        pos, taken = st
        v = pv[0, pl.ds(j * 16, 16)]
        bits = plsc.bitcast(v, jnp.int32)
        m_gt = bits > tstar
        m_eq = bits == tstar
        eqc = plsc.cumsum(jnp.where(m_eq, 1, 0))
        take_eq = m_eq & ((eqc + taken) <= need)
        selm = m_gt | take_eq
        idxs = lax.iota(jnp.int32, 16) + j * 16
        plsc.store_compressed(selb.at[pl.ds(pos, 16)], idxs, mask=selm)
        plsc.store_compressed(wvb.at[pl.ds(pos, 16)], v, mask=selm)
        return (pos + jnp.sum(jnp.where(selm, 1, 0)),
                taken + jnp.sum(jnp.where(take_eq, 1, 0)))
    lax.fori_loop(0, _S // 16, _comp, (jnp.int32(0), jnp.int32(0)))

    pltpu.sync_copy(selb.at[pl.ds(0, _TOPK)], sel_ref.at[c, s, 0])
    pltpu.sync_copy(wvb.at[pl.ds(0, _TOPK)], w_ref.at[c, s, 0])
    for h in range(2):
        pltpu.async_copy(x_ref.at[c].at[selb.at[pl.ds(h * 64, 64)]],
                         rows_g, sem).wait()
        pltpu.sync_copy(rows_g, xg_ref.at[c, s, pl.ds(h * 64, 64)])


def _topk_gather(probt, x):
    mesh = plsc.VectorSubcoreMesh(core_axis_name="c", subcore_axis_name="s",
                                  num_cores=_NC, num_subcores=_NS)
    f = pl.kernel(
        _topk_gather_body,
        out_type=[
            jax.ShapeDtypeStruct((_B, _E, 1, _TOPK), jnp.int32),
            jax.ShapeDtypeStruct((_B, _E, 1, _TOPK), jnp.float32),
            jax.ShapeDtypeStruct((_B, _E, _TOPK, _DIM), jnp.float32),
        ],
        mesh=mesh,
        scratch_types=[
            pltpu.VMEM((1, _S), jnp.float32),
            pltpu.VMEM((160,), jnp.int32),
            pltpu.VMEM((160,), jnp.float32),
            pltpu.VMEM((64, _DIM), jnp.float32),
            pltpu.SemaphoreType.DMA,
        ],
        compiler_params=pltpu.CompilerParams(needs_layout_passes=False),
    )
    return f(probt.reshape(_B, _E, 1, _S), x)


def _scatter_body(contrib_ref, sel_ref, out_ref, out_v, cbuf, idx_v):
    c = lax.axis_index("c")
    s = lax.axis_index("s")
    wid = s * _NC + c
    col0 = wid * _TC
    for b in range(_B):
        @plsc.parallel_loop(0, _TC * _S // 16, unroll=8)
        def _zero(i):
            out_v[pl.ds(i * 16, 16)] = jnp.zeros((16,), jnp.float32)

        pltpu.sync_copy(sel_ref.at[b], idx_v)

        def _sub(sub, _):
            pltpu.sync_copy(
                contrib_ref.at[b, pl.ds(col0, _TC), pl.ds(sub * _RC, _RC)],
                cbuf)
            for r4 in range(_RC // _TOPK):
                e = sub * (_RC // _TOPK) + r4
                cb = r4 * _TOPK
                for k in range(_TOPK // 16):
                    idxg = idx_v[e, pl.ds(k * 16, 16)]
                    for colc in range(_TC):
                        x = cbuf[colc, pl.ds(cb + k * 16, 16)]
                        plsc.addupdate_scatter(
                            out_v, [idxg + (colc * _S)], x)
            return 0
        lax.fori_loop(0, _E * _TOPK // _RC, _sub, 0)
        for colc in range(_TC):
            pltpu.sync_copy(out_v.at[pl.ds(colc * _S, _S)],
                            out_ref.at[b, col0 + colc, :])


def _scatter_add(contrib_t, sel):
    mesh = plsc.VectorSubcoreMesh(core_axis_name="c", subcore_axis_name="s",
                                  num_cores=_NC, num_subcores=_NS)
    f = pl.kernel(
        _scatter_body,
        out_type=jax.ShapeDtypeStruct((_B, _DIM, _S), jnp.float32),
        mesh=mesh,
        scratch_types=[
            pltpu.VMEM((_TC * _S,), jnp.float32),
            pltpu.VMEM((_TC, _RC), jnp.float32),
            pltpu.VMEM((_E, _TOPK), jnp.int32),
        ],
        compiler_params=pltpu.CompilerParams(use_tc_tiling_on_sc=False,
                                             needs_layout_passes=False),
    )
    return f(contrib_t, sel)


def kernel(inputs, router_w, fc1_w, fc1_b, ln_g, ln_b, fc2_w, fc2_b):
    router_logits, probt = _router(inputs, router_w)
    selected, weights, xg = _topk_gather(probt, inputs)
    selected = selected.reshape(_B, _E, _TOPK)

    contrib_t = _ffn(xg.reshape(_B * _E, _TOPK, _DIM),
                     weights.reshape(_B * _E, 1, _TOPK),
                     fc1_w, fc1_b.reshape(_E, 1, _FFN),
                     ln_g.reshape(_E, 1, _FFN), ln_b.reshape(_E, 1, _FFN),
                     fc2_w, fc2_b.reshape(_E, 1, _DIM))

    out_t = _scatter_add(contrib_t, selected)
    return jnp.swapaxes(out_t, 1, 2), router_logits


# speedup vs baseline: 1.1460x; 1.0228x over previous
import functools
import jax
import jax.numpy as jnp
from jax import lax
from jax.experimental import pallas as pl
from jax.experimental.pallas import tpu as pltpu
from jax.experimental.pallas import tpu_sc as plsc

_B, _S, _DIM = 2, 2048, 1024
_E, _FFN, _TOPK = 16, 2048, 128
_EPS = 1e-5


def _router_body(x_ref, rw_ref, logits_ref, probt_ref):
    x = x_ref[0]
    rw = rw_ref[...]
    logits = lax.dot_general(x, rw, (((1,), (1,)), ((), ())),
                             preferred_element_type=jnp.float32)
    logits_ref[0] = logits
    m = jnp.max(logits, axis=-1, keepdims=True)
    ex = jnp.exp(logits - m)
    probs = ex / jnp.sum(ex, axis=-1, keepdims=True)
    probt_ref[0] = probs.T


def _router(inputs, router_w):
    return pl.pallas_call(
        _router_body,
        grid=(_B,),
        in_specs=[
            pl.BlockSpec((1, _S, _DIM), lambda b: (b, 0, 0)),
            pl.BlockSpec((_E, _DIM), lambda b: (0, 0)),
        ],
        out_specs=[
            pl.BlockSpec((1, _S, _E), lambda b: (b, 0, 0)),
            pl.BlockSpec((1, _E, _S), lambda b: (b, 0, 0)),
        ],
        out_shape=[
            jax.ShapeDtypeStruct((_B, _S, _E), jnp.float32),
            jax.ShapeDtypeStruct((_B, _E, _S), jnp.float32),
        ],
    )(inputs, router_w)


def _ffn_body(xg_ref, w_ref, fc1w_ref, fc1b_ref, lng_ref, lnb_ref,
              fc2w_ref, fc2b_ref, out_ref):
    xg = xg_ref[0].astype(jnp.bfloat16)
    h = lax.dot_general(xg, fc1w_ref[0].astype(jnp.bfloat16),
                        (((1,), (1,)), ((), ())),
                        preferred_element_type=jnp.float32)
    h = h + fc1b_ref[0]
    h = 0.5 * h * (1.0 + lax.erf(h * 0.7071067811865476))
    mu = jnp.mean(h, axis=-1, keepdims=True)
    var = jnp.mean(jnp.square(h - mu), axis=-1, keepdims=True)
    h = (h - mu) * lax.rsqrt(var + _EPS) * lng_ref[0] + lnb_ref[0]
    y = lax.dot_general(fc2w_ref[0].astype(jnp.bfloat16),
                        h.astype(jnp.bfloat16),
                        (((1,), (1,)), ((), ())),
                        preferred_element_type=jnp.float32)
    y = y + fc2b_ref[0].reshape(_DIM, 1)
    out_ref[0] = y * w_ref[0]


def _ffn(xg, wsel, fc1_w, fc1_b, ln_g, ln_b, fc2_w, fc2_b):
    grid = (_B * _E,)
    return pl.pallas_call(
        _ffn_body,
        grid=grid,
        in_specs=[
            pl.BlockSpec((1, _TOPK, _DIM), lambda i: (i, 0, 0)),
            pl.BlockSpec((1, 1, _TOPK), lambda i: (i, 0, 0)),
            pl.BlockSpec((1, _FFN, _DIM), lambda i: (i % _E, 0, 0)),
            pl.BlockSpec((1, 1, _FFN), lambda i: (i % _E, 0, 0)),
            pl.BlockSpec((1, 1, _FFN), lambda i: (i % _E, 0, 0)),
            pl.BlockSpec((1, 1, _FFN), lambda i: (i % _E, 0, 0)),
            pl.BlockSpec((1, _DIM, _FFN), lambda i: (i % _E, 0, 0)),
            pl.BlockSpec((1, 1, _DIM), lambda i: (i % _E, 0, 0)),
        ],
        out_specs=pl.BlockSpec((1, _DIM, _TOPK), lambda i: (i // _E, 0, i % _E)),
        out_shape=jax.ShapeDtypeStruct((_B, _DIM, _E * _TOPK), jnp.float32),
    )(xg, wsel, fc1_w, fc1_b, ln_g, ln_b, fc2_w, fc2_b)


_NC, _NS = 2, 16
_CH = _S // _NS
_NW = _NC * _NS
_TC = _DIM // _NW
_RC = 512


def _topk_gather_body(probt_ref, x_ref, sel_ref, w_ref, xg_ref,
                      pv, selb, wvb, rows_g, sem):
    c = lax.axis_index("c")
    s = lax.axis_index("s")
    pltpu.sync_copy(probt_ref.at[c, s], pv)

    def _count(thr):
        def cbody(j, acc):
            for u in range(4):
                bits = plsc.bitcast(pv[0, pl.ds(j * 64 + u * 16, 16)],
                                    jnp.int32)
                acc = acc + jnp.where(bits >= thr, 1, 0)
            return acc
        acc = lax.fori_loop(0, _S // 64, cbody,
                            jnp.zeros((16,), jnp.int32))
        return jnp.sum(acc)

    def _bis(i, lohi):
        lo, hi = lohi
        mid = (lo + hi) // 2
        big = _count(mid) >= _TOPK
        return (jnp.where(big, mid, lo), jnp.where(big, hi, mid))
    tstar, _ = lax.fori_loop(0, 31, _bis,
                             (jnp.int32(0), jnp.int32(0x3F800001)))
    need = _TOPK - _count(tstar + 1)

    def _comp(j, st):
        pos, taken = st
        v = pv[0, pl.ds(j * 16, 16)]
        bits = plsc.bitcast(v, jnp.int32)
        m_gt = bits > tstar
        m_eq = bits == tstar
        eqc = plsc.cumsum(jnp.where(m_eq, 1, 0))
        take_eq = m_eq & ((eqc + taken) <= need)
        selm = m_gt | take_eq
        idxs = lax.iota(jnp.int32, 16) + j * 16
        plsc.store_compressed(selb.at[pl.ds(pos, 16)], idxs, mask=selm)
        plsc.store_compressed(wvb.at[pl.ds(pos, 16)], v, mask=selm)
        return (pos + jnp.sum(jnp.where(selm, 1, 0)),
                taken + jnp.sum(jnp.where(take_eq, 1, 0)))
    lax.fori_loop(0, _S // 16, _comp, (jnp.int32(0), jnp.int32(0)))

    pltpu.sync_copy(selb.at[pl.ds(0, _TOPK)], sel_ref.at[c, s, 0])
    pltpu.sync_copy(wvb.at[pl.ds(0, _TOPK)], w_ref.at[c, s, 0])
    for h in range(2):
        pltpu.async_copy(x_ref.at[c].at[selb.at[pl.ds(h * 64, 64)]],
                         rows_g, sem).wait()
        pltpu.sync_copy(rows_g, xg_ref.at[c, s, pl.ds(h * 64, 64)])


def _topk_gather(probt, x):
    mesh = plsc.VectorSubcoreMesh(core_axis_name="c", subcore_axis_name="s",
                                  num_cores=_NC, num_subcores=_NS)
    f = pl.kernel(
        _topk_gather_body,
        out_type=[
            jax.ShapeDtypeStruct((_B, _E, 1, _TOPK), jnp.int32),
            jax.ShapeDtypeStruct((_B, _E, 1, _TOPK), jnp.float32),
            jax.ShapeDtypeStruct((_B, _E, _TOPK, _DIM), jnp.float32),
        ],
        mesh=mesh,
        scratch_types=[
            pltpu.VMEM((1, _S), jnp.float32),
            pltpu.VMEM((160,), jnp.int32),
            pltpu.VMEM((160,), jnp.float32),
            pltpu.VMEM((64, _DIM), jnp.float32),
            pltpu.SemaphoreType.DMA,
        ],
        compiler_params=pltpu.CompilerParams(needs_layout_passes=False),
    )
    return f(probt.reshape(_B, _E, 1, _S), x)


def _scatter_body(contrib_ref, sel_ref, out_ref, out_v, cbuf, idx_v):
    c = lax.axis_index("c")
    s = lax.axis_index("s")
    wid = s * _NC + c
    col0 = wid * _TC
    for b in range(_B):
        @plsc.parallel_loop(0, _TC * _S // 16, unroll=8)
        def _zero(i):
            out_v[pl.ds(i * 16, 16)] = jnp.zeros((16,), jnp.float32)

        pltpu.sync_copy(sel_ref.at[b], idx_v)

        def _sub(sub, _):
            pltpu.sync_copy(
                contrib_ref.at[b, pl.ds(col0, _TC), pl.ds(sub * _RC, _RC)],
                cbuf)
            for r4 in range(_RC // _TOPK):
                e = sub * (_RC // _TOPK) + r4
                cb = r4 * _TOPK
                for k in range(_TOPK // 16):
                    idxg = idx_v[e, pl.ds(k * 16, 16)]
                    for colc in range(_TC):
                        x = cbuf[colc, pl.ds(cb + k * 16, 16)]
                        plsc.addupdate_scatter(
                            out_v, [idxg + (colc * _S)], x)
            return 0
        lax.fori_loop(0, _E * _TOPK // _RC, _sub, 0)
        for colc in range(_TC):
            pltpu.sync_copy(out_v.at[pl.ds(colc * _S, _S)],
                            out_ref.at[b, col0 + colc, :])


def _scatter_add(contrib_t, sel):
    mesh = plsc.VectorSubcoreMesh(core_axis_name="c", subcore_axis_name="s",
                                  num_cores=_NC, num_subcores=_NS)
    f = pl.kernel(
        _scatter_body,
        out_type=jax.ShapeDtypeStruct((_B, _DIM, _S), jnp.float32),
        mesh=mesh,
        scratch_types=[
            pltpu.VMEM((_TC * _S,), jnp.float32),
            pltpu.VMEM((_TC, _RC), jnp.float32),
            pltpu.VMEM((_E, _TOPK), jnp.int32),
        ],
        compiler_params=pltpu.CompilerParams(use_tc_tiling_on_sc=False,
                                             needs_layout_passes=False),
    )
    return f(contrib_t, sel)


def kernel(inputs, router_w, fc1_w, fc1_b, ln_g, ln_b, fc2_w, fc2_b):
    router_logits, probt = _router(inputs, router_w)
    selected, weights, xg = _topk_gather(probt, inputs)
    selected = selected.reshape(_B, _E, _TOPK)

    contrib_t = _ffn(xg.reshape(_B * _E, _TOPK, _DIM),
                     weights.reshape(_B * _E, 1, _TOPK),
                     fc1_w, fc1_b.reshape(_E, 1, _FFN),
                     ln_g.reshape(_E, 1, _FFN), ln_b.reshape(_E, 1, _FFN),
                     fc2_w, fc2_b.reshape(_E, 1, _DIM))

    out_t = _scatter_add(contrib_t, selected)
    return jnp.swapaxes(out_t, 1, 2), router_logits
